# baseline (speedup 1.0000x reference)
import jax
import jax.numpy as jnp
from jax import lax
from jax.experimental import pallas as pl
from jax.experimental.pallas import tpu as pltpu
from jax.experimental.pallas import tpu_sc as plsc

SEQ = 2048
DIM = 256
NC = 2
NS = 16
NW = NC * NS
TOTAL = 4 * SEQ
ROWS_PER_W = TOTAL // NW
CH = 64
NCHUNK = ROWS_PER_W // CH
LANES = 16
DCHUNKS = DIM // LANES


def _emb_body(x_hbm, tok_hbm, pos_hbm, out_hbm, idx_v, tok_v, pos_v,
              gat_sem, pos_sem, st_sem):
    wid = lax.axis_index("s") * NC + lax.axis_index("c")
    base = wid * ROWS_PER_W
    pos0 = lax.rem(base, SEQ)

    for j in range(NCHUNK):
        pltpu.sync_copy(x_hbm.at[pl.ds(base + j * CH, CH)], idx_v.at[j])
    pos_cp = pltpu.async_copy(pos_hbm.at[pl.ds(pos0, ROWS_PER_W)], pos_v,
                              pos_sem)

    gathers = [None] * NCHUNK
    stores = [None] * NCHUNK
    gathers[0] = pltpu.async_copy(tok_hbm.at[idx_v.at[0]], tok_v.at[0],
                                  gat_sem)

    for j in range(NCHUNK):
        b = j % 2
        if j + 1 < NCHUNK:
            if j >= 1:
                stores[j - 1].wait()
            gathers[j + 1] = pltpu.async_copy(
                tok_hbm.at[idx_v.at[j + 1]], tok_v.at[1 - b], gat_sem)
        gathers[j].wait()
        if j == 0:
            pos_cp.wait()

        @plsc.parallel_loop(0, CH, unroll=4)
        def _add(r):
            for c in range(DCHUNKS):
                sl = pl.ds(c * LANES, LANES)
                tok_v[b, r, sl] = tok_v[b, r, sl] + pos_v[j * CH + r, sl]

        stores[j] = pltpu.async_copy(
            tok_v.at[b], out_hbm.at[pl.ds(base + j * CH, CH)], st_sem)

    stores[NCHUNK - 2].wait()
    stores[NCHUNK - 1].wait()


def kernel(x, token_table, pos_table):
    B, S = x.shape
    xf = x.reshape(B * S).astype(jnp.int32)
    call = pl.kernel(
        _emb_body,
        out_type=jax.ShapeDtypeStruct((B * S, DIM), jnp.float32),
        mesh=plsc.VectorSubcoreMesh(core_axis_name="c", subcore_axis_name="s"),
        scratch_types=[
            pltpu.VMEM((NCHUNK, CH), jnp.int32),
            pltpu.VMEM((2, CH, DIM), jnp.float32),
            pltpu.VMEM((ROWS_PER_W, DIM), jnp.float32),
            pltpu.SemaphoreType.DMA,
            pltpu.SemaphoreType.DMA,
            pltpu.SemaphoreType.DMA,
        ],
    )
    out = call(xf, token_table, pos_table)
    return out.reshape(B, S, DIM)


# speedup vs baseline: 1.2515x; 1.2515x over previous
import jax
import jax.numpy as jnp
from jax import lax
from jax.experimental import pallas as pl
from jax.experimental.pallas import tpu as pltpu
from jax.experimental.pallas import tpu_sc as plsc

SEQ = 2048
DIM = 256
BATCH = 4
NC = 2
NS = 16
NW = NC * NS
S_PER_W = SEQ // NW
HALF = S_PER_W // 2
LANES = 16
DCHUNKS = DIM // LANES


def _emb_body(x_hbm, tok_hbm, pos_hbm, out_hbm, idx_v, tok_v, pos_v,
              idx_sem, pos_sem, gat_sem, st_sem):
    wid = lax.axis_index("s") * NC + lax.axis_index("c")
    s0 = wid * S_PER_W

    idx_cps = [
        pltpu.async_copy(x_hbm.at[pl.ds(b * SEQ + s0, S_PER_W)],
                         idx_v.at[b], idx_sem)
        for b in range(BATCH)
    ]
    pos_cp = pltpu.async_copy(pos_hbm.at[pl.ds(s0, S_PER_W)], pos_v, pos_sem)

    gathers = [None] * BATCH
    for b in range(BATCH):
        idx_cps[b].wait()
        gathers[b] = pltpu.async_copy(
            tok_hbm.at[idx_v.at[b]], tok_v.at[b], gat_sem)
    pos_cp.wait()
    for b in range(BATCH):
        gathers[b].wait()

    @plsc.parallel_loop(0, S_PER_W)
    def _add(r):
        for c in range(DCHUNKS):
            sl = pl.ds(c * LANES, LANES)
            p = pos_v[r, sl]
            for b in range(BATCH):
                tok_v[b, r, sl] = tok_v[b, r, sl] + p

    stores = []
    for b in range(BATCH):
        stores.append(pltpu.async_copy(
            tok_v.at[b], out_hbm.at[pl.ds(b * SEQ + s0, S_PER_W)], st_sem))
    for st in stores:
        st.wait()


def kernel(x, token_table, pos_table):
    B, S = x.shape
    xf = x.reshape(B * S).astype(jnp.int32)
    call = pl.kernel(
        _emb_body,
        out_type=jax.ShapeDtypeStruct((B * S, DIM), jnp.float32),
        mesh=plsc.VectorSubcoreMesh(core_axis_name="c", subcore_axis_name="s"),
        scratch_types=[
            pltpu.VMEM((BATCH, S_PER_W), jnp.int32),
            pltpu.VMEM((BATCH, S_PER_W, DIM), jnp.float32),
            pltpu.VMEM((S_PER_W, DIM), jnp.float32),
            pltpu.SemaphoreType.DMA,
            pltpu.SemaphoreType.DMA,
            pltpu.SemaphoreType.DMA,
            pltpu.SemaphoreType.DMA,
        ],
    )
    out = call(xf, token_table, pos_table)
    return out.reshape(B, S, DIM)


# speedup vs baseline: 1.2544x; 1.0023x over previous
import jax
import jax.numpy as jnp
from jax import lax
from jax.experimental import pallas as pl
from jax.experimental.pallas import tpu as pltpu
from jax.experimental.pallas import tpu_sc as plsc

SEQ = 2048
DIM = 256
BATCH = 4
NC = 2
NS = 16
NW = NC * NS
S_PER_W = SEQ // NW
HALF = S_PER_W // 2
LANES = 16
DCHUNKS = DIM // LANES


def _emb_body(x_hbm, tok_hbm, pos_hbm, out_hbm, idx_v, tok_v, pos_v,
              idx_sem, pos_sem, gat_sem, st_sem):
    wid = lax.axis_index("s") * NC + lax.axis_index("c")
    s0 = wid * S_PER_W

    idx_cps = [
        pltpu.async_copy(x_hbm.at[pl.ds(b * SEQ + s0, S_PER_W)],
                         idx_v.at[b], idx_sem)
        for b in range(BATCH)
    ]
    pos_cp = pltpu.async_copy(pos_hbm.at[pl.ds(s0, S_PER_W)], pos_v, pos_sem)

    gathers = [None] * BATCH
    for b in range(BATCH):
        idx_cps[b].wait()
        gathers[b] = pltpu.async_copy(
            tok_hbm.at[idx_v.at[b]], tok_v.at[b], gat_sem)
    pos_cp.wait()
    for b in range(BATCH):
        gathers[b].wait()

    @plsc.parallel_loop(0, S_PER_W)
    def _add(r):
        for c in range(DCHUNKS):
            sl = pl.ds(c * LANES, LANES)
            p = pos_v[r, sl]
            for b in range(BATCH):
                plsc.addupdate(tok_v.at[b, r, sl], p)

    stores = []
    for b in range(BATCH):
        stores.append(pltpu.async_copy(
            tok_v.at[b], out_hbm.at[pl.ds(b * SEQ + s0, S_PER_W)], st_sem))
    for st in stores:
        st.wait()


def kernel(x, token_table, pos_table):
    B, S = x.shape
    xf = x.reshape(B * S).astype(jnp.int32)
    call = pl.kernel(
        _emb_body,
        out_type=jax.ShapeDtypeStruct((B * S, DIM), jnp.float32),
        mesh=plsc.VectorSubcoreMesh(core_axis_name="c", subcore_axis_name="s"),
        scratch_types=[
            pltpu.VMEM((BATCH, S_PER_W), jnp.int32),
            pltpu.VMEM((BATCH, S_PER_W, DIM), jnp.float32),
            pltpu.VMEM((S_PER_W, DIM), jnp.float32),
            pltpu.SemaphoreType.DMA,
            pltpu.SemaphoreType.DMA,
            pltpu.SemaphoreType.DMA,
            pltpu.SemaphoreType.DMA,
        ],
    )
    out = call(xf, token_table, pos_table)
    return out.reshape(B, S, DIM)


# speedup vs baseline: 1.3226x; 1.0544x over previous
import jax
import jax.numpy as jnp
from jax import lax
from jax.experimental import pallas as pl
from jax.experimental.pallas import tpu as pltpu
from jax.experimental.pallas import tpu_sc as plsc

SEQ = 2048
DIM = 256
BATCH = 4
NC = 2
NS = 16
NW = NC * NS
S_PER_W = SEQ // NW
NH = 2
HALF = S_PER_W // NH
LANES = 16
DCHUNKS = DIM // LANES


def _emb_body(x_hbm, tok_hbm, pos_hbm, out_hbm, idx_v, tok_v, pos_v,
              idx_sem, pos_sem, gat_sem, st_sem):
    wid = lax.axis_index("s") * NC + lax.axis_index("c")
    s0 = wid * S_PER_W

    idx_cps = [
        pltpu.async_copy(x_hbm.at[pl.ds(b * SEQ + s0 + h * HALF, HALF)],
                         idx_v.at[h * BATCH + b], idx_sem)
        for h in range(NH) for b in range(BATCH)
    ]
    pos_cp = pltpu.async_copy(pos_hbm.at[pl.ds(s0, S_PER_W)], pos_v, pos_sem)

    gathers = [None] * (NH * BATCH)
    for k in range(NH * BATCH):
        idx_cps[k].wait()
        gathers[k] = pltpu.async_copy(
            tok_hbm.at[idx_v.at[k]], tok_v.at[k], gat_sem)
    pos_cp.wait()

    stores = []
    for h in range(NH):
        for b in range(BATCH):
            gathers[h * BATCH + b].wait()

        @plsc.parallel_loop(0, HALF)
        def _add(r):
            for c in range(DCHUNKS):
                sl = pl.ds(c * LANES, LANES)
                p = pos_v[h * HALF + r, sl]
                for b in range(BATCH):
                    plsc.addupdate(tok_v.at[h * BATCH + b, r, sl], p)

        for b in range(BATCH):
            stores.append(pltpu.async_copy(
                tok_v.at[h * BATCH + b],
                out_hbm.at[pl.ds(b * SEQ + s0 + h * HALF, HALF)], st_sem))
    for st in stores:
        st.wait()


def kernel(x, token_table, pos_table):
    B, S = x.shape
    xf = x.reshape(B * S).astype(jnp.int32)
    call = pl.kernel(
        _emb_body,
        out_type=jax.ShapeDtypeStruct((B * S, DIM), jnp.float32),
        mesh=plsc.VectorSubcoreMesh(core_axis_name="c", subcore_axis_name="s"),
        scratch_types=[
            pltpu.VMEM((NH * BATCH, HALF), jnp.int32),
            pltpu.VMEM((NH * BATCH, HALF, DIM), jnp.float32),
            pltpu.VMEM((S_PER_W, DIM), jnp.float32),
            pltpu.SemaphoreType.DMA,
            pltpu.SemaphoreType.DMA,
            pltpu.SemaphoreType.DMA,
            pltpu.SemaphoreType.DMA,
        ],
    )
    out = call(xf, token_table, pos_table)
    return out.reshape(B, S, DIM)
